# baseline (speedup 1.0000x reference)
import functools

import jax
import jax.numpy as jnp
from jax import lax
from jax.experimental import pallas as pl
from jax.experimental.pallas import tpu as pltpu
from jax.experimental.pallas import tpu_sc as plsc

C = 256
N = 10000
K = 16
NP = 10240
NT = 320
G = 4
NW = 32
L = 16


def _tc_body(x_ref, w_ref, a_ref, wh_ref, s_ref):
    xb = x_ref[...]
    wt = w_ref[...]
    whb = lax.dot_general(xb, wt, (((0,), (1,)), ((), ())),
                          preferred_element_type=jnp.float32)
    wh_ref[...] = whb
    aw = a_ref[...]
    s_ref[...] = lax.dot_general(aw, whb, (((1,), (1,)), ((), ())),
                                 preferred_element_type=jnp.float32)


def _sc_body(wh_hbm, s1_hbm, s2_hbm, idx0_hbm, idx1_hbm, out_hbm,
             s1_v, s2_v, idx0_v, idx1_v, rows_a, rows_b, out_a, out_b, tab_sh,
             sema, semb, semoa, semob):
    nc = 2
    sid = lax.axis_index("s")
    wid = sid * nc + lax.axis_index("c")
    node_base = wid * NT
    eoff = node_base * K
    ng = NT // G

    rows_per_sub = NP // 16
    pltpu.sync_copy(wh_hbm.at[pl.ds(sid * rows_per_sub, rows_per_sub)],
                    tab_sh.at[pl.ds(sid * rows_per_sub, rows_per_sub)])

    pltpu.sync_copy(s1_hbm, s1_v)
    pltpu.sync_copy(s2_hbm, s2_v)
    pltpu.sync_copy(idx0_hbm.at[pl.ds(eoff, NT * K)], idx0_v)
    pltpu.sync_copy(idx1_hbm.at[pl.ds(eoff, NT * K)], idx1_v)
    plsc.subcore_barrier()

    def issue(g, rows, sem):
        off = jnp.minimum(g, ng - 1) * (G * K)
        pltpu.async_copy(tab_sh.at[idx0_v.at[pl.ds(off, G * K)]], rows, sem)

    def drain(rows, sem):
        pltpu.make_async_copy(wh_hbm.at[pl.ds(0, G * K)], rows, sem).wait()

    def wait_out(out_v, semo):
        pltpu.make_async_copy(out_v, out_hbm.at[pl.ds(0, G)], semo).wait()

    def compute(g, rows, out_v, semo):
        for j in range(G):
            off = g * (G * K) + j * K
            i1 = idx1_v[pl.ds(off, K)]
            i0 = idx0_v[pl.ds(off, K)]
            e = plsc.load_gather(s1_v, [i1]) + plsc.load_gather(s2_v, [i0])
            e = jnp.maximum(e, 0.2 * e)
            e = jnp.exp(e - jnp.max(e))
            a = e / jnp.sum(e)
            acc = [jnp.zeros((L,), jnp.float32) for _ in range(C // L)]
            for k in range(K):
                ak = a[k]
                r = j * K + k
                for c in range(C // (2 * L)):
                    ch = rows[r, pl.ds(c * 2 * L, 2 * L)]
                    u, v = plsc.unpack(ch, format=plsc.PackFormat.INTERLEAVED)
                    acc[2 * c] = acc[2 * c] + ak * u
                    acc[2 * c + 1] = acc[2 * c + 1] + ak * v
            for c in range(C // L):
                out_v[j, pl.ds(c * L, L)] = acc[c]
        pltpu.async_copy(out_v, out_hbm.at[pl.ds(node_base + g * G, G)], semo)

    issue(0, rows_a, sema)
    issue(1, rows_b, semb)

    def pair(gp, _):
        g0 = 2 * gp
        drain(rows_a, sema)

        @pl.when(gp > 0)
        def _():
            wait_out(out_a, semoa)
            wait_out(out_b, semob)

        compute(g0, rows_a, out_a, semoa)
        issue(g0 + 2, rows_a, sema)
        drain(rows_b, semb)
        compute(g0 + 1, rows_b, out_b, semob)
        issue(g0 + 3, rows_b, semb)
        return _

    lax.fori_loop(0, ng // 2, pair, None)
    wait_out(out_a, semoa)
    wait_out(out_b, semob)
    drain(rows_a, sema)
    drain(rows_b, semb)


@jax.jit
def kernel(x, edge_index, W, a_w):
    x2 = x[0, :, :, 0]
    x2p = jnp.pad(x2, ((0, 0), (0, NP - N)))
    aw8 = jnp.zeros((8, C), jnp.float32)
    aw8 = aw8.at[0].set(a_w[:C]).at[1].set(a_w[C:])

    bn = 1280
    wh_rows, s8 = pl.pallas_call(
        _tc_body,
        grid=(NP // bn,),
        in_specs=[
            pl.BlockSpec((C, bn), lambda i: (0, i)),
            pl.BlockSpec((C, C), lambda i: (0, 0)),
            pl.BlockSpec((8, C), lambda i: (0, 0)),
        ],
        out_specs=[
            pl.BlockSpec((bn, C), lambda i: (i, 0)),
            pl.BlockSpec((8, bn), lambda i: (0, i)),
        ],
        out_shape=[
            jax.ShapeDtypeStruct((NP, C), jnp.float32),
            jax.ShapeDtypeStruct((8, NP), jnp.float32),
        ],
    )(x2p, W, aw8)

    idx0 = jnp.pad(edge_index[0, 0], ((0, NP - N), (0, 0))).reshape(-1)
    idx1 = jnp.pad(edge_index[1, 0], ((0, NP - N), (0, 0))).reshape(-1)

    wh_bf = (wh_rows.reshape(NP, C // 32, 2, 16)
             .transpose(0, 1, 3, 2).reshape(NP, C).astype(jnp.bfloat16))

    sc = pl.kernel(
        _sc_body,
        out_type=jax.ShapeDtypeStruct((NP, C), jnp.float32),
        mesh=plsc.VectorSubcoreMesh(core_axis_name="c", subcore_axis_name="s",
                                    num_cores=2, num_subcores=16),
        compiler_params=pltpu.CompilerParams(needs_layout_passes=False,
                                             use_tc_tiling_on_sc=False),
        scratch_types=[
            pltpu.VMEM((NP,), jnp.float32),
            pltpu.VMEM((NP,), jnp.float32),
            pltpu.VMEM((NT * K,), jnp.int32),
            pltpu.VMEM((NT * K,), jnp.int32),
            pltpu.VMEM((G * K, C), jnp.bfloat16),
            pltpu.VMEM((G * K, C), jnp.bfloat16),
            pltpu.VMEM((G, C), jnp.float32),
            pltpu.VMEM((G, C), jnp.float32),
            pltpu.VMEM_SHARED((NP, C), jnp.bfloat16),
            pltpu.SemaphoreType.DMA,
            pltpu.SemaphoreType.DMA,
            pltpu.SemaphoreType.DMA,
            pltpu.SemaphoreType.DMA,
        ],
    )
    out_rows = sc(wh_bf, s8[0], s8[1], idx0, idx1)

    return out_rows[:N].T[None, :, :, None]


# speedup vs baseline: 1.2954x; 1.2954x over previous
import functools

import jax
import jax.numpy as jnp
from jax import lax
from jax.experimental import pallas as pl
from jax.experimental.pallas import tpu as pltpu
from jax.experimental.pallas import tpu_sc as plsc

C = 256
N = 10000
K = 16
NP = 10240
NT = 320
G = 2
NW = 32
L = 16


def _tc_body(x_ref, w_ref, a_ref, wh_ref, s_ref):
    xb = x_ref[...]
    wt = w_ref[...]
    whb = lax.dot_general(xb, wt, (((0,), (1,)), ((), ())),
                          preferred_element_type=jnp.float32)
    wh_ref[...] = whb
    aw = a_ref[...]
    s_ref[...] = lax.dot_general(aw, whb, (((1,), (1,)), ((), ())),
                                 preferred_element_type=jnp.float32)


def _sc_body(wh_hbm, s1_hbm, s2_hbm, idx0_hbm, idx1_hbm, out_hbm,
             s1_v, s2_v, idx0_v, idx1_v, rows_a, rows_b, out_a, out_b, tab_sh,
             sema, semb, semoa, semob):
    nc = 2
    sid = lax.axis_index("s")
    wid = sid * nc + lax.axis_index("c")
    node_base = wid * NT
    eoff = node_base * K
    ng = NT // G

    rows_per_sub = NP // 16
    pltpu.sync_copy(wh_hbm.at[pl.ds(sid * rows_per_sub, rows_per_sub)],
                    tab_sh.at[pl.ds(sid * rows_per_sub, rows_per_sub)])

    pltpu.sync_copy(s1_hbm, s1_v)
    pltpu.sync_copy(s2_hbm, s2_v)
    pltpu.sync_copy(idx0_hbm.at[pl.ds(eoff, NT * K)], idx0_v)
    pltpu.sync_copy(idx1_hbm.at[pl.ds(eoff, NT * K)], idx1_v)
    plsc.subcore_barrier()

    def issue(g, rows, sem):
        off = jnp.minimum(g, ng - 1) * (G * K)
        pltpu.async_copy(tab_sh.at[idx0_v.at[pl.ds(off, G * K)]], rows, sem)

    def drain(rows, sem):
        pltpu.make_async_copy(wh_hbm.at[pl.ds(0, G * K)], rows, sem).wait()

    def wait_out(out_v, semo):
        pltpu.make_async_copy(out_v, out_hbm.at[pl.ds(0, G)], semo).wait()

    def compute(g, rows, out_v, semo):
        for j in range(G):
            off = g * (G * K) + j * K
            i1 = idx1_v[pl.ds(off, K)]
            i0 = idx0_v[pl.ds(off, K)]
            e = plsc.load_gather(s1_v, [i1]) + plsc.load_gather(s2_v, [i0])
            e = jnp.maximum(e, 0.2 * e)
            e = jnp.exp(e - jnp.max(e))
            a = e / jnp.sum(e)
            acc = [jnp.zeros((L,), jnp.float32) for _ in range(C // L)]
            for k in range(K):
                ak = a[k]
                r = j * K + k
                for c in range(C // (2 * L)):
                    ch = rows[r, pl.ds(c * 2 * L, 2 * L)]
                    u, v = plsc.unpack(ch, format=plsc.PackFormat.INTERLEAVED)
                    acc[2 * c] = acc[2 * c] + ak * u
                    acc[2 * c + 1] = acc[2 * c + 1] + ak * v
            for c in range(C // L):
                out_v[j, pl.ds(c * L, L)] = acc[c]
        pltpu.async_copy(out_v, out_hbm.at[pl.ds(node_base + g * G, G)], semo)

    issue(0, rows_a, sema)
    issue(1, rows_b, semb)

    def pair(gp, _):
        g0 = 2 * gp
        drain(rows_a, sema)

        @pl.when(gp > 0)
        def _():
            wait_out(out_a, semoa)
            wait_out(out_b, semob)

        compute(g0, rows_a, out_a, semoa)
        issue(g0 + 2, rows_a, sema)
        drain(rows_b, semb)
        compute(g0 + 1, rows_b, out_b, semob)
        issue(g0 + 3, rows_b, semb)
        return _

    lax.fori_loop(0, ng // 2, pair, None)
    wait_out(out_a, semoa)
    wait_out(out_b, semob)
    drain(rows_a, sema)
    drain(rows_b, semb)


@jax.jit
def kernel(x, edge_index, W, a_w):
    x2 = x[0, :, :, 0]
    x2p = jnp.pad(x2, ((0, 0), (0, NP - N)))
    aw8 = jnp.zeros((8, C), jnp.float32)
    aw8 = aw8.at[0].set(a_w[:C]).at[1].set(a_w[C:])

    bn = 1280
    wh_rows, s8 = pl.pallas_call(
        _tc_body,
        grid=(NP // bn,),
        in_specs=[
            pl.BlockSpec((C, bn), lambda i: (0, i)),
            pl.BlockSpec((C, C), lambda i: (0, 0)),
            pl.BlockSpec((8, C), lambda i: (0, 0)),
        ],
        out_specs=[
            pl.BlockSpec((bn, C), lambda i: (i, 0)),
            pl.BlockSpec((8, bn), lambda i: (0, i)),
        ],
        out_shape=[
            jax.ShapeDtypeStruct((NP, C), jnp.float32),
            jax.ShapeDtypeStruct((8, NP), jnp.float32),
        ],
    )(x2p, W, aw8)

    idx0 = jnp.pad(edge_index[0, 0], ((0, NP - N), (0, 0))).reshape(-1)
    idx1 = jnp.pad(edge_index[1, 0], ((0, NP - N), (0, 0))).reshape(-1)

    wh_bf = (wh_rows.reshape(NP, C // 32, 2, 16)
             .transpose(0, 1, 3, 2).reshape(NP, C).astype(jnp.bfloat16))

    sc = pl.kernel(
        _sc_body,
        out_type=jax.ShapeDtypeStruct((NP, C), jnp.float32),
        mesh=plsc.VectorSubcoreMesh(core_axis_name="c", subcore_axis_name="s",
                                    num_cores=2, num_subcores=16),
        compiler_params=pltpu.CompilerParams(needs_layout_passes=False,
                                             use_tc_tiling_on_sc=False),
        scratch_types=[
            pltpu.VMEM((NP,), jnp.float32),
            pltpu.VMEM((NP,), jnp.float32),
            pltpu.VMEM((NT * K,), jnp.int32),
            pltpu.VMEM((NT * K,), jnp.int32),
            pltpu.VMEM((G * K, C), jnp.bfloat16),
            pltpu.VMEM((G * K, C), jnp.bfloat16),
            pltpu.VMEM((G, C), jnp.float32),
            pltpu.VMEM((G, C), jnp.float32),
            pltpu.VMEM_SHARED((NP, C), jnp.bfloat16),
            pltpu.SemaphoreType.DMA,
            pltpu.SemaphoreType.DMA,
            pltpu.SemaphoreType.DMA,
            pltpu.SemaphoreType.DMA,
        ],
    )
    out_rows = sc(wh_bf, s8[0], s8[1], idx0, idx1)

    return out_rows[:N].T[None, :, :, None]


# speedup vs baseline: 1.6565x; 1.2788x over previous
import functools

import jax
import jax.numpy as jnp
import numpy as np
from jax import lax
from jax.experimental import pallas as pl
from jax.experimental.pallas import tpu as pltpu
from jax.experimental.pallas import tpu_sc as plsc

C = 256
N = 10000
K = 16
NP = 10240
NT = 320
G = 2
NW = 32
L = 16


def _tc_body(x_ref, w_ref, a_ref, wh_ref, s_ref):
    xb = x_ref[...]
    wt = w_ref[...]
    whb = lax.dot_general(xb, wt, (((0,), (1,)), ((), ())),
                          preferred_element_type=jnp.float32)
    wh_ref[...] = whb.astype(jnp.bfloat16)
    aw = a_ref[...]
    s_ref[...] = lax.dot_general(aw, whb, (((1,), (1,)), ((), ())),
                                 preferred_element_type=jnp.float32)


def _sc_body(wh_hbm, s1_hbm, s2_hbm, idx0_hbm, idx1_hbm, out_hbm,
             s1_v, s2_v, idx0_v, idx1_v, rows_a, rows_b, out_a, out_b, tab_sh,
             sema, semb, semoa, semob):
    nc = 2
    sid = lax.axis_index("s")
    wid = sid * nc + lax.axis_index("c")
    node_base = wid * NT
    eoff = node_base * K
    ng = NT // G

    rows_per_sub = NP // 16
    pltpu.sync_copy(wh_hbm.at[pl.ds(sid * rows_per_sub, rows_per_sub)],
                    tab_sh.at[pl.ds(sid * rows_per_sub, rows_per_sub)])

    pltpu.sync_copy(s1_hbm, s1_v)
    pltpu.sync_copy(s2_hbm, s2_v)
    pltpu.sync_copy(idx0_hbm.at[pl.ds(eoff, NT * K)], idx0_v)
    pltpu.sync_copy(idx1_hbm.at[pl.ds(eoff, NT * K)], idx1_v)
    plsc.subcore_barrier()

    def issue(g, rows, sem):
        off = jnp.minimum(g, ng - 1) * (G * K)
        pltpu.async_copy(tab_sh.at[idx0_v.at[pl.ds(off, G * K)]], rows, sem)

    def drain(rows, sem):
        pltpu.make_async_copy(wh_hbm.at[pl.ds(0, G * K)], rows, sem).wait()

    def wait_out(out_v, semo):
        pltpu.make_async_copy(out_v, out_hbm.at[pl.ds(0, G)], semo).wait()

    def compute(g, rows, out_v, semo):
        for j in range(G):
            off = g * (G * K) + j * K
            i1 = idx1_v[pl.ds(off, K)]
            i0 = idx0_v[pl.ds(off, K)]
            e = plsc.load_gather(s1_v, [i1]) + plsc.load_gather(s2_v, [i0])
            e = jnp.maximum(e, 0.2 * e)
            e = jnp.exp(e - jnp.max(e))
            a = e / jnp.sum(e)
            acc = [jnp.zeros((L,), jnp.float32) for _ in range(C // L)]
            for k in range(K):
                ak = a[k]
                r = j * K + k
                for c in range(C // (2 * L)):
                    ch = rows[r, pl.ds(c * 2 * L, 2 * L)]
                    u, v = plsc.unpack(ch, format=plsc.PackFormat.INTERLEAVED)
                    acc[2 * c] = acc[2 * c] + ak * u
                    acc[2 * c + 1] = acc[2 * c + 1] + ak * v
            for c in range(C // L):
                out_v[j, pl.ds(c * L, L)] = acc[c]
        pltpu.async_copy(out_v, out_hbm.at[pl.ds(node_base + g * G, G)], semo)

    issue(0, rows_a, sema)
    issue(1, rows_b, semb)

    def pair(gp, _):
        g0 = 2 * gp
        drain(rows_a, sema)

        @pl.when(gp > 0)
        def _():
            wait_out(out_a, semoa)
            wait_out(out_b, semob)

        compute(g0, rows_a, out_a, semoa)
        issue(g0 + 2, rows_a, sema)
        drain(rows_b, semb)
        compute(g0 + 1, rows_b, out_b, semob)
        issue(g0 + 3, rows_b, semb)
        return _

    lax.fori_loop(0, ng // 2, pair, None)
    wait_out(out_a, semoa)
    wait_out(out_b, semob)
    drain(rows_a, sema)
    drain(rows_b, semb)


@jax.jit
def kernel(x, edge_index, W, a_w):
    x2 = x[0, :, :, 0]

    perm = np.arange(C).reshape(C // 32, 2, 16).transpose(0, 2, 1).reshape(-1)
    wp = W[perm, :]
    aw8 = jnp.zeros((8, C), jnp.float32)
    aw8 = aw8.at[0].set(a_w[:C][perm]).at[1].set(a_w[C:][perm])

    bn = 1280
    wh_bf, s8 = pl.pallas_call(
        _tc_body,
        grid=(NP // bn,),
        in_specs=[
            pl.BlockSpec((C, bn), lambda i: (0, i)),
            pl.BlockSpec((C, C), lambda i: (0, 0)),
            pl.BlockSpec((8, C), lambda i: (0, 0)),
        ],
        out_specs=[
            pl.BlockSpec((bn, C), lambda i: (i, 0)),
            pl.BlockSpec((8, bn), lambda i: (0, i)),
        ],
        out_shape=[
            jax.ShapeDtypeStruct((NP, C), jnp.bfloat16),
            jax.ShapeDtypeStruct((8, NP), jnp.float32),
        ],
    )(x2, wp, aw8)

    idx0 = jnp.pad(edge_index[0, 0], ((0, NP - N), (0, 0))).reshape(-1)
    idx1 = jnp.pad(edge_index[1, 0], ((0, NP - N), (0, 0))).reshape(-1)


    sc = pl.kernel(
        _sc_body,
        out_type=jax.ShapeDtypeStruct((NP, C), jnp.float32),
        mesh=plsc.VectorSubcoreMesh(core_axis_name="c", subcore_axis_name="s",
                                    num_cores=2, num_subcores=16),
        compiler_params=pltpu.CompilerParams(needs_layout_passes=False,
                                             use_tc_tiling_on_sc=False),
        scratch_types=[
            pltpu.VMEM((NP,), jnp.float32),
            pltpu.VMEM((NP,), jnp.float32),
            pltpu.VMEM((NT * K,), jnp.int32),
            pltpu.VMEM((NT * K,), jnp.int32),
            pltpu.VMEM((G * K, C), jnp.bfloat16),
            pltpu.VMEM((G * K, C), jnp.bfloat16),
            pltpu.VMEM((G, C), jnp.float32),
            pltpu.VMEM((G, C), jnp.float32),
            pltpu.VMEM_SHARED((NP, C), jnp.bfloat16),
            pltpu.SemaphoreType.DMA,
            pltpu.SemaphoreType.DMA,
            pltpu.SemaphoreType.DMA,
            pltpu.SemaphoreType.DMA,
        ],
    )
    out_rows = sc(wh_bf, s8[0], s8[1], idx0, idx1)

    return out_rows[:N].T[None, :, :, None]
